# baseline (speedup 1.0000x reference)
import functools

import jax
import jax.numpy as jnp
from jax import lax
from jax.experimental import pallas as pl
from jax.experimental.pallas import tpu as pltpu
from jax.experimental.pallas import tpu_sc as plsc

N = 10000
D = 128
E = 320000

NC = 2
NS = 16
CHUNK = 128
NCH_T = 80
EP = NC * NS * NCH_T * CHUNK
TRASH = N

NPAD = 10112
ROWS_T = NPAD // NS
HPAD = 10240
HROWS_T = HPAD // NS


def _mesh():
    return plsc.VectorSubcoreMesh(core_axis_name="c", subcore_axis_name="s")


def _sc_degrees(s0, d0, s1, d1):
    mesh = _mesh()

    @functools.partial(
        pl.kernel,
        out_type=jax.ShapeDtypeStruct((NC * 4 * HPAD,), jnp.float32),
        mesh=mesh,
        compiler_params=pltpu.CompilerParams(needs_layout_passes=False),
        scratch_types=[
            pltpu.VMEM((NCH_T, CHUNK), jnp.int32),
            pltpu.VMEM((4 * HPAD,), jnp.float32),
            pltpu.VMEM((HROWS_T,), jnp.float32),
            pltpu.VMEM((HROWS_T,), jnp.float32),
            pltpu.VMEM_SHARED((NS * 4 * HPAD,), jnp.float32),
        ],
    )
    def k(s0r, d0r, s1r, d1r, out, idx_v, lhist, stg, acc, shared):
        c = lax.axis_index("c")
        s = lax.axis_index("s")
        g = c * NS + s

        zero16 = jnp.zeros((16,), jnp.float32)
        ones16 = jnp.ones((16,), jnp.float32)

        def zbody(i, _):
            lhist[pl.ds(i * 16, 16)] = zero16
            return 0

        lax.fori_loop(0, 4 * HPAD // 16, zbody, 0)

        for a, arr in enumerate((s0r, d0r, s1r, d1r)):
            pltpu.sync_copy(arr.at[pl.ds(g * NCH_T, NCH_T)], idx_v)

            def hbody(j, _, a=a):
                for kk in range(CHUNK // 16):
                    iv = idx_v[j, pl.ds(kk * 16, 16)] + (a * HPAD)
                    plsc.addupdate_scatter(lhist, [iv], ones16)
                return 0

            lax.fori_loop(0, NCH_T, hbody, 0)

        pltpu.sync_copy(lhist, shared.at[pl.ds(s * 4 * HPAD, 4 * HPAD)])
        plsc.subcore_barrier()

        for a in range(4):

            def zacc(i, _):
                acc[pl.ds(i * 16, 16)] = zero16
                return 0

            lax.fori_loop(0, HROWS_T // 16, zacc, 0)

            for t in range(NS):
                pltpu.sync_copy(
                    shared.at[pl.ds(t * 4 * HPAD + a * HPAD + s * HROWS_T,
                                    HROWS_T)], stg)

                def radd(i, _):
                    sl = pl.ds(i * 16, 16)
                    acc[sl] = acc[sl] + stg[sl]
                    return 0

                lax.fori_loop(0, HROWS_T // 16, radd, 0)

            off = (c * 4 + a) * HPAD + s * HROWS_T
            pltpu.sync_copy(acc, out.at[pl.ds(off, HROWS_T)])

    return k(s0, d0, s1, d1)


HALF = 40
GRP = 2
CH0 = 160
CH1 = 2 * NCH_T - CH0


def _sc_agg(xs0, xs1, s0, d0, s1, d1, zrows):
    mesh = _mesh()

    @functools.partial(
        pl.kernel,
        out_type=jax.ShapeDtypeStruct((4 * NPAD, D), jnp.float32),
        mesh=mesh,
        scratch_types=[
            pltpu.VMEM((HALF, CHUNK), jnp.int32),
            pltpu.VMEM((HALF, CHUNK), jnp.int32),
            pltpu.VMEM((CHUNK, D), jnp.float32),
            pltpu.VMEM((CHUNK, D), jnp.float32),
            pltpu.SemaphoreType.DMA,
            pltpu.SemaphoreType.DMA,
            pltpu.VMEM_SHARED((NPAD, D), jnp.float32),
        ],
    )
    def k(xs0r, xs1r, s0r, d0r, s1r, d1r, zr, out,
          sidx, didx, bufa, bufb, sema, semb, agg):
        bufs = (bufa, bufb)
        sems = (sema, semb)
        c = lax.axis_index("c")
        s = lax.axis_index("s")

        def run(tile_base, nch):
            for r, (xsr, sr, dr) in enumerate(
                    ((xs0r, s0r, d0r), (xs1r, s1r, d1r))):
                pltpu.sync_copy(zr, agg.at[pl.ds(s * ROWS_T, ROWS_T)])
                plsc.subcore_barrier()

                for h in range(nch // HALF):
                    base = tile_base + h * HALF
                    pltpu.sync_copy(sr.at[pl.ds(base, HALF)], sidx)
                    pltpu.sync_copy(dr.at[pl.ds(base, HALF)], didx)

                    def body(gg, _, xsr=xsr):
                        cps = []
                        for u in range(GRP):
                            j = gg * GRP + u
                            cps.append(pltpu.async_copy(
                                xsr.at[sidx.at[j]], bufs[u], sems[u]))
                        for u in range(GRP):
                            j = gg * GRP + u
                            cps[u].wait()
                            pltpu.sync_copy(bufs[u], agg.at[didx.at[j]],
                                            add=True)
                        return 0

                    lax.fori_loop(0, HALF // GRP, body, 0)
                plsc.subcore_barrier()

                off = (r * NC + c) * NPAD + s * ROWS_T
                pltpu.sync_copy(agg.at[pl.ds(s * ROWS_T, ROWS_T)],
                                out.at[pl.ds(off, ROWS_T)])
                if r == 0:
                    plsc.subcore_barrier()

        @pl.when(c == 0)
        def _():
            run(s * CH0, CH0)

        @pl.when(c == 1)
        def _():
            run(NS * CH0 + s * CH1, CH1)

    return k(xs0, xs1, s0, d0, s1, d1, zrows)


def _tc_scale(x, ht):
    def body(x_ref, ht_ref, xs0_ref, xs1_ref):
        xv = x_ref[...]
        od0 = ht_ref[:, 0:1] + ht_ref[:, 4:5]
        od1 = ht_ref[:, 2:3] + ht_ref[:, 6:7]
        isr0 = lax.rsqrt(jnp.maximum(od0, 1.0))[:N]
        isr1 = lax.rsqrt(jnp.maximum(od1, 1.0))[:N]
        xs0_ref[:N] = xv * isr0
        xs1_ref[:N] = xv * isr1
        zpad = jnp.zeros((NPAD - N, D), jnp.float32)
        xs0_ref[N:] = zpad
        xs1_ref[N:] = zpad

    return pl.pallas_call(
        body,
        out_shape=[
            jax.ShapeDtypeStruct((NPAD, D), jnp.float32),
            jax.ShapeDtypeStruct((NPAD, D), jnp.float32),
        ],
    )(x, ht)


def _tc_final(aggp, ht, W0, b0, W1, b1):
    def body(aggp_ref, ht_ref, w0_ref, b0_ref, w1_ref, b1_ref, out_ref):
        a0 = aggp_ref[0, :N, :] + aggp_ref[1, :N, :]
        a1 = aggp_ref[2, :N, :] + aggp_ref[3, :N, :]
        id0 = ht_ref[:, 1:2] + ht_ref[:, 5:6]
        id1 = ht_ref[:, 3:4] + ht_ref[:, 7:8]
        isr0 = lax.rsqrt(jnp.maximum(id0, 1.0))[:N]
        isr1 = lax.rsqrt(jnp.maximum(id1, 1.0))[:N]
        r0 = a0 * isr0
        r1 = a1 * isr1
        acc = jnp.dot(r0, w0_ref[...], preferred_element_type=jnp.float32)
        acc += jnp.dot(r1, w1_ref[...], preferred_element_type=jnp.float32)
        out_ref[...] = acc + (b0_ref[...] + b1_ref[...])[None, :]

    return pl.pallas_call(
        body,
        out_shape=jax.ShapeDtypeStruct((N, D), jnp.float32),
    )(aggp, ht, W0, b0, W1, b1)


def kernel(x, edge_index_rel0, edge_index_rel1, W0, b0, W1, b1):
    pad = jnp.full((EP - E,), TRASH, jnp.int32)

    def prep(v):
        return jnp.concatenate([v.astype(jnp.int32), pad]).reshape(
            NC * NS * NCH_T, CHUNK)

    s0 = prep(edge_index_rel0[0])
    d0 = prep(edge_index_rel0[1])
    s1 = prep(edge_index_rel1[0])
    d1 = prep(edge_index_rel1[1])

    zrows = jnp.zeros((ROWS_T, D), jnp.float32)

    hists = _sc_degrees(s0, d0, s1, d1)
    ht = hists.reshape(NC * 4, HPAD).T

    xs0, xs1 = _tc_scale(x, ht)

    aggp = _sc_agg(xs0, xs1, s0, d0, s1, d1, zrows)
    aggp = aggp.reshape(4, NPAD, D)

    return _tc_final(aggp, ht, W0, b0, W1, b1)


# speedup vs baseline: 1.4151x; 1.4151x over previous
import functools

import jax
import jax.numpy as jnp
from jax import lax
from jax.experimental import pallas as pl
from jax.experimental.pallas import tpu as pltpu
from jax.experimental.pallas import tpu_sc as plsc

N = 10000
D = 128
E = 320000

NC = 2
NS = 16
CHUNK = 128
NCH_T = 80
EP = NC * NS * NCH_T * CHUNK
TRASH = N

NPAD = 10112
ROWS_T = NPAD // NS
HPAD = 10240
HROWS_T = HPAD // NS


def _mesh():
    return plsc.VectorSubcoreMesh(core_axis_name="c", subcore_axis_name="s")


def _sc_degrees(s0, d0, s1, d1):
    mesh = _mesh()

    @functools.partial(
        pl.kernel,
        out_type=jax.ShapeDtypeStruct((NC * 4 * HPAD,), jnp.float32),
        mesh=mesh,
        compiler_params=pltpu.CompilerParams(needs_layout_passes=False),
        scratch_types=[
            pltpu.VMEM((NCH_T, CHUNK), jnp.int32),
            pltpu.VMEM((4 * HPAD,), jnp.float32),
            pltpu.VMEM((HROWS_T,), jnp.float32),
            pltpu.VMEM((HROWS_T,), jnp.float32),
            pltpu.VMEM_SHARED((NS * 4 * HPAD,), jnp.float32),
        ],
    )
    def k(s0r, d0r, s1r, d1r, out, idx_v, lhist, stg, acc, shared):
        c = lax.axis_index("c")
        s = lax.axis_index("s")
        g = c * NS + s

        zero16 = jnp.zeros((16,), jnp.float32)
        ones16 = jnp.ones((16,), jnp.float32)

        def zbody(i, _):
            lhist[pl.ds(i * 16, 16)] = zero16
            return 0

        lax.fori_loop(0, 4 * HPAD // 16, zbody, 0)

        for a, arr in enumerate((s0r, d0r, s1r, d1r)):
            pltpu.sync_copy(arr.at[pl.ds(g * NCH_T, NCH_T)], idx_v)

            def hbody(j, _, a=a):
                for kk in range(CHUNK // 16):
                    iv = idx_v[j, pl.ds(kk * 16, 16)] + (a * HPAD)
                    plsc.addupdate_scatter(lhist, [iv], ones16)
                return 0

            lax.fori_loop(0, NCH_T, hbody, 0)

        pltpu.sync_copy(lhist, shared.at[pl.ds(s * 4 * HPAD, 4 * HPAD)])
        plsc.subcore_barrier()

        for a in range(4):

            def zacc(i, _):
                acc[pl.ds(i * 16, 16)] = zero16
                return 0

            lax.fori_loop(0, HROWS_T // 16, zacc, 0)

            for t in range(NS):
                pltpu.sync_copy(
                    shared.at[pl.ds(t * 4 * HPAD + a * HPAD + s * HROWS_T,
                                    HROWS_T)], stg)

                def radd(i, _):
                    sl = pl.ds(i * 16, 16)
                    acc[sl] = acc[sl] + stg[sl]
                    return 0

                lax.fori_loop(0, HROWS_T // 16, radd, 0)

            off = (c * 4 + a) * HPAD + s * HROWS_T
            pltpu.sync_copy(acc, out.at[pl.ds(off, HROWS_T)])

    return k(s0, d0, s1, d1)


HALF = 16
GRP = 2
CH0 = 112
CH1 = 2 * NCH_T - CH0


def _sc_agg(xs0, xs1, s0, d0, s1, d1, zrows):
    mesh = _mesh()

    @functools.partial(
        pl.kernel,
        out_type=jax.ShapeDtypeStruct((4 * NPAD, D), jnp.float32),
        mesh=mesh,
        scratch_types=[
            pltpu.VMEM((HALF, CHUNK), jnp.int32),
            pltpu.VMEM((HALF, CHUNK), jnp.int32),
            pltpu.VMEM((CHUNK, D), jnp.float32),
            pltpu.VMEM((CHUNK, D), jnp.float32),
            pltpu.SemaphoreType.DMA,
            pltpu.SemaphoreType.DMA,
            pltpu.VMEM_SHARED((NPAD, D), jnp.float32),
        ],
    )
    def k(xs0r, xs1r, s0r, d0r, s1r, d1r, zr, out,
          sidx, didx, bufa, bufb, sema, semb, agg):
        bufs = (bufa, bufb)
        sems = (sema, semb)
        c = lax.axis_index("c")
        s = lax.axis_index("s")

        def run(tile_base, nch):
            for r, (xsr, sr, dr) in enumerate(
                    ((xs0r, s0r, d0r), (xs1r, s1r, d1r))):
                pltpu.sync_copy(zr, agg.at[pl.ds(s * ROWS_T, ROWS_T)])
                plsc.subcore_barrier()

                for h in range(nch // HALF):
                    base = tile_base + h * HALF
                    pltpu.sync_copy(sr.at[pl.ds(base, HALF)], sidx)
                    pltpu.sync_copy(dr.at[pl.ds(base, HALF)], didx)

                    def body(gg, _, xsr=xsr):
                        cps = []
                        for u in range(GRP):
                            j = gg * GRP + u
                            cps.append(pltpu.async_copy(
                                xsr.at[sidx.at[j]], bufs[u], sems[u]))
                        for u in range(GRP):
                            j = gg * GRP + u
                            cps[u].wait()
                            pltpu.sync_copy(bufs[u], agg.at[didx.at[j]],
                                            add=True)
                        return 0

                    lax.fori_loop(0, HALF // GRP, body, 0)
                plsc.subcore_barrier()

                off = (r * NC + c) * NPAD + s * ROWS_T
                pltpu.sync_copy(agg.at[pl.ds(s * ROWS_T, ROWS_T)],
                                out.at[pl.ds(off, ROWS_T)])
                if r == 0:
                    plsc.subcore_barrier()

        @pl.when(c == 0)
        def _():
            run(s * CH0, CH0)

        @pl.when(c == 1)
        def _():
            run(NS * CH0 + s * CH1, CH1)

    return k(xs0, xs1, s0, d0, s1, d1, zrows)


def _tc_scale(x, ht):
    def body(x_ref, ht_ref, xs0_ref, xs1_ref):
        xv = x_ref[...]
        od0 = ht_ref[:, 0:1] + ht_ref[:, 4:5]
        od1 = ht_ref[:, 2:3] + ht_ref[:, 6:7]
        isr0 = lax.rsqrt(jnp.maximum(od0, 1.0))[:N]
        isr1 = lax.rsqrt(jnp.maximum(od1, 1.0))[:N]
        xs0_ref[:N] = xv * isr0
        xs1_ref[:N] = xv * isr1
        zpad = jnp.zeros((NPAD - N, D), jnp.float32)
        xs0_ref[N:] = zpad
        xs1_ref[N:] = zpad

    return pl.pallas_call(
        body,
        out_shape=[
            jax.ShapeDtypeStruct((NPAD, D), jnp.float32),
            jax.ShapeDtypeStruct((NPAD, D), jnp.float32),
        ],
    )(x, ht)


def _tc_final(aggp, ht, W0, b0, W1, b1):
    def body(aggp_ref, ht_ref, w0_ref, b0_ref, w1_ref, b1_ref, out_ref):
        a0 = aggp_ref[0, :N, :] + aggp_ref[1, :N, :]
        a1 = aggp_ref[2, :N, :] + aggp_ref[3, :N, :]
        id0 = ht_ref[:, 1:2] + ht_ref[:, 5:6]
        id1 = ht_ref[:, 3:4] + ht_ref[:, 7:8]
        isr0 = lax.rsqrt(jnp.maximum(id0, 1.0))[:N]
        isr1 = lax.rsqrt(jnp.maximum(id1, 1.0))[:N]
        r0 = a0 * isr0
        r1 = a1 * isr1
        acc = jnp.dot(r0, w0_ref[...], preferred_element_type=jnp.float32)
        acc += jnp.dot(r1, w1_ref[...], preferred_element_type=jnp.float32)
        out_ref[...] = acc + (b0_ref[...] + b1_ref[...])[None, :]

    return pl.pallas_call(
        body,
        out_shape=jax.ShapeDtypeStruct((N, D), jnp.float32),
    )(aggp, ht, W0, b0, W1, b1)


def kernel(x, edge_index_rel0, edge_index_rel1, W0, b0, W1, b1):
    pad = jnp.full((EP - E,), TRASH, jnp.int32)

    def prep(v):
        return jnp.concatenate([v.astype(jnp.int32), pad]).reshape(
            NC * NS * NCH_T, CHUNK)

    s0 = prep(edge_index_rel0[0])
    d0 = prep(edge_index_rel0[1])
    s1 = prep(edge_index_rel1[0])
    d1 = prep(edge_index_rel1[1])

    zrows = jnp.zeros((ROWS_T, D), jnp.float32)

    hists = _sc_degrees(s0, d0, s1, d1)
    ht = hists.reshape(NC * 4, HPAD).T

    xs0, xs1 = _tc_scale(x, ht)

    aggp = _sc_agg(xs0, xs1, s0, d0, s1, d1, zrows)
    aggp = aggp.reshape(4, NPAD, D)

    return _tc_final(aggp, ht, W0, b0, W1, b1)


# speedup vs baseline: 1.5142x; 1.0700x over previous
import functools

import jax
import jax.numpy as jnp
from jax import lax
from jax.experimental import pallas as pl
from jax.experimental.pallas import tpu as pltpu
from jax.experimental.pallas import tpu_sc as plsc

N = 10000
D = 128
E = 320000

NC = 2
NS = 16
CHUNK = 128
NCH_T = 80
EP = NC * NS * NCH_T * CHUNK
TRASH = N

NPAD = 10112
ROWS_T = NPAD // NS
HPAD = 10240
HROWS_T = HPAD // NS


def _mesh():
    return plsc.VectorSubcoreMesh(core_axis_name="c", subcore_axis_name="s")


def _sc_degrees(s0, d0, s1, d1):
    mesh = _mesh()

    @functools.partial(
        pl.kernel,
        out_type=jax.ShapeDtypeStruct((NC * 4 * HPAD,), jnp.float32),
        mesh=mesh,
        compiler_params=pltpu.CompilerParams(needs_layout_passes=False),
        scratch_types=[
            pltpu.VMEM((NCH_T, CHUNK), jnp.int32),
            pltpu.VMEM((4 * HPAD,), jnp.float32),
            pltpu.VMEM((HROWS_T,), jnp.float32),
            pltpu.VMEM((HROWS_T,), jnp.float32),
            pltpu.VMEM_SHARED((NS * 4 * HPAD,), jnp.float32),
        ],
    )
    def k(s0r, d0r, s1r, d1r, out, idx_v, lhist, stg, acc, shared):
        c = lax.axis_index("c")
        s = lax.axis_index("s")
        g = c * NS + s

        zero16 = jnp.zeros((16,), jnp.float32)
        ones16 = jnp.ones((16,), jnp.float32)

        def zbody(i, _):
            lhist[pl.ds(i * 16, 16)] = zero16
            return 0

        lax.fori_loop(0, 4 * HPAD // 16, zbody, 0)

        for a, arr in enumerate((s0r, d0r, s1r, d1r)):
            pltpu.sync_copy(arr.at[pl.ds(g * NCH_T, NCH_T)], idx_v)

            def hbody(j, _, a=a):
                for kk in range(CHUNK // 16):
                    iv = idx_v[j, pl.ds(kk * 16, 16)] + (a * HPAD)
                    plsc.addupdate_scatter(lhist, [iv], ones16)
                return 0

            lax.fori_loop(0, NCH_T, hbody, 0)

        pltpu.sync_copy(lhist, shared.at[pl.ds(s * 4 * HPAD, 4 * HPAD)])
        plsc.subcore_barrier()

        for a in range(4):

            def zacc(i, _):
                acc[pl.ds(i * 16, 16)] = zero16
                return 0

            lax.fori_loop(0, HROWS_T // 16, zacc, 0)

            for t in range(NS):
                pltpu.sync_copy(
                    shared.at[pl.ds(t * 4 * HPAD + a * HPAD + s * HROWS_T,
                                    HROWS_T)], stg)

                def radd(i, _):
                    sl = pl.ds(i * 16, 16)
                    acc[sl] = acc[sl] + stg[sl]
                    return 0

                lax.fori_loop(0, HROWS_T // 16, radd, 0)

            off = (c * 4 + a) * HPAD + s * HROWS_T
            pltpu.sync_copy(acc, out.at[pl.ds(off, HROWS_T)])

    return k(s0, d0, s1, d1)


HALF = 16
GRP = 2
CH0 = 128
CH1 = 2 * NCH_T - CH0


def _sc_agg(xs0, xs1, s0, d0, s1, d1, zrows):
    mesh = _mesh()

    @functools.partial(
        pl.kernel,
        out_type=jax.ShapeDtypeStruct((4 * NPAD, D), jnp.float32),
        mesh=mesh,
        scratch_types=[
            pltpu.VMEM((HALF, CHUNK), jnp.int32),
            pltpu.VMEM((HALF, CHUNK), jnp.int32),
            pltpu.VMEM((CHUNK, D), jnp.float32),
            pltpu.VMEM((CHUNK, D), jnp.float32),
            pltpu.SemaphoreType.DMA,
            pltpu.SemaphoreType.DMA,
            pltpu.VMEM_SHARED((NPAD, D), jnp.float32),
        ],
    )
    def k(xs0r, xs1r, s0r, d0r, s1r, d1r, zr, out,
          sidx, didx, bufa, bufb, sema, semb, agg):
        bufs = (bufa, bufb)
        sems = (sema, semb)
        c = lax.axis_index("c")
        s = lax.axis_index("s")

        def run(tile_base, nch):
            for r, (xsr, sr, dr) in enumerate(
                    ((xs0r, s0r, d0r), (xs1r, s1r, d1r))):
                pltpu.sync_copy(zr, agg.at[pl.ds(s * ROWS_T, ROWS_T)])
                plsc.subcore_barrier()

                for h in range(nch // HALF):
                    base = tile_base + h * HALF
                    pltpu.sync_copy(sr.at[pl.ds(base, HALF)], sidx)
                    pltpu.sync_copy(dr.at[pl.ds(base, HALF)], didx)

                    def body(gg, _, xsr=xsr):
                        cps = []
                        for u in range(GRP):
                            j = gg * GRP + u
                            cps.append(pltpu.async_copy(
                                xsr.at[sidx.at[j]], bufs[u], sems[u]))
                        for u in range(GRP):
                            j = gg * GRP + u
                            cps[u].wait()
                            pltpu.sync_copy(bufs[u], agg.at[didx.at[j]],
                                            add=True)
                        return 0

                    lax.fori_loop(0, HALF // GRP, body, 0)
                plsc.subcore_barrier()

                off = (r * NC + c) * NPAD + s * ROWS_T
                pltpu.sync_copy(agg.at[pl.ds(s * ROWS_T, ROWS_T)],
                                out.at[pl.ds(off, ROWS_T)])
                if r == 0:
                    plsc.subcore_barrier()

        @pl.when(c == 0)
        def _():
            run(s * CH0, CH0)

        @pl.when(c == 1)
        def _():
            run(NS * CH0 + s * CH1, CH1)

    return k(xs0, xs1, s0, d0, s1, d1, zrows)


def _tc_scale(x, ht):
    def body(x_ref, ht_ref, xs0_ref, xs1_ref):
        xv = x_ref[...]
        od0 = ht_ref[:, 0:1] + ht_ref[:, 4:5]
        od1 = ht_ref[:, 2:3] + ht_ref[:, 6:7]
        isr0 = lax.rsqrt(jnp.maximum(od0, 1.0))[:N]
        isr1 = lax.rsqrt(jnp.maximum(od1, 1.0))[:N]
        xs0_ref[:N] = xv * isr0
        xs1_ref[:N] = xv * isr1
        zpad = jnp.zeros((NPAD - N, D), jnp.float32)
        xs0_ref[N:] = zpad
        xs1_ref[N:] = zpad

    return pl.pallas_call(
        body,
        out_shape=[
            jax.ShapeDtypeStruct((NPAD, D), jnp.float32),
            jax.ShapeDtypeStruct((NPAD, D), jnp.float32),
        ],
    )(x, ht)


def _tc_final(aggp, ht, W0, b0, W1, b1):
    def body(aggp_ref, ht_ref, w0_ref, b0_ref, w1_ref, b1_ref, out_ref):
        a0 = aggp_ref[0, :N, :] + aggp_ref[1, :N, :]
        a1 = aggp_ref[2, :N, :] + aggp_ref[3, :N, :]
        id0 = ht_ref[:, 1:2] + ht_ref[:, 5:6]
        id1 = ht_ref[:, 3:4] + ht_ref[:, 7:8]
        isr0 = lax.rsqrt(jnp.maximum(id0, 1.0))[:N]
        isr1 = lax.rsqrt(jnp.maximum(id1, 1.0))[:N]
        r0 = a0 * isr0
        r1 = a1 * isr1
        acc = jnp.dot(r0, w0_ref[...], preferred_element_type=jnp.float32)
        acc += jnp.dot(r1, w1_ref[...], preferred_element_type=jnp.float32)
        out_ref[...] = acc + (b0_ref[...] + b1_ref[...])[None, :]

    return pl.pallas_call(
        body,
        out_shape=jax.ShapeDtypeStruct((N, D), jnp.float32),
    )(aggp, ht, W0, b0, W1, b1)


def kernel(x, edge_index_rel0, edge_index_rel1, W0, b0, W1, b1):
    pad = jnp.full((EP - E,), TRASH, jnp.int32)

    def prep(v):
        return jnp.concatenate([v.astype(jnp.int32), pad]).reshape(
            NC * NS * NCH_T, CHUNK)

    s0 = prep(edge_index_rel0[0])
    d0 = prep(edge_index_rel0[1])
    s1 = prep(edge_index_rel1[0])
    d1 = prep(edge_index_rel1[1])

    zrows = jnp.zeros((ROWS_T, D), jnp.float32)

    hists = _sc_degrees(s0, d0, s1, d1)
    ht = hists.reshape(NC * 4, HPAD).T

    xs0, xs1 = _tc_scale(x, ht)

    aggp = _sc_agg(xs0, xs1, s0, d0, s1, d1, zrows)
    aggp = aggp.reshape(4, NPAD, D)

    return _tc_final(aggp, ht, W0, b0, W1, b1)


# speedup vs baseline: 1.5945x; 1.0530x over previous
import functools

import jax
import jax.numpy as jnp
from jax import lax
from jax.experimental import pallas as pl
from jax.experimental.pallas import tpu as pltpu
from jax.experimental.pallas import tpu_sc as plsc

N = 10000
D = 128
E = 320000

NC = 2
NS = 16
CHUNK = 128
NCH_T = 80
EP = NC * NS * NCH_T * CHUNK
TRASH = N

NPAD = 10112
ROWS_T = NPAD // NS
HPAD = 10240
HROWS_T = HPAD // NS


def _mesh():
    return plsc.VectorSubcoreMesh(core_axis_name="c", subcore_axis_name="s")


def _sc_degrees(s0, d0, s1, d1):
    mesh = _mesh()

    @functools.partial(
        pl.kernel,
        out_type=jax.ShapeDtypeStruct((NC * 4 * HPAD,), jnp.float32),
        mesh=mesh,
        compiler_params=pltpu.CompilerParams(needs_layout_passes=False),
        scratch_types=[
            pltpu.VMEM((NCH_T, CHUNK), jnp.int32),
            pltpu.VMEM((4 * HPAD,), jnp.float32),
            pltpu.VMEM((HROWS_T,), jnp.float32),
            pltpu.VMEM((HROWS_T,), jnp.float32),
            pltpu.VMEM_SHARED((NS * 4 * HPAD,), jnp.float32),
        ],
    )
    def k(s0r, d0r, s1r, d1r, out, idx_v, lhist, stg, acc, shared):
        c = lax.axis_index("c")
        s = lax.axis_index("s")
        g = c * NS + s

        zero16 = jnp.zeros((16,), jnp.float32)
        ones16 = jnp.ones((16,), jnp.float32)

        def zbody(i, _):
            lhist[pl.ds(i * 16, 16)] = zero16
            return 0

        lax.fori_loop(0, 4 * HPAD // 16, zbody, 0)

        for a, arr in enumerate((s0r, d0r, s1r, d1r)):
            pltpu.sync_copy(arr.at[pl.ds(g * NCH_T, NCH_T)], idx_v)

            def hbody(j, _, a=a):
                for kk in range(CHUNK // 16):
                    iv = idx_v[j, pl.ds(kk * 16, 16)] + (a * HPAD)
                    plsc.addupdate_scatter(lhist, [iv], ones16)
                return 0

            lax.fori_loop(0, NCH_T, hbody, 0)

        pltpu.sync_copy(lhist, shared.at[pl.ds(s * 4 * HPAD, 4 * HPAD)])
        plsc.subcore_barrier()

        for a in range(4):

            def zacc(i, _):
                acc[pl.ds(i * 16, 16)] = zero16
                return 0

            lax.fori_loop(0, HROWS_T // 16, zacc, 0)

            for t in range(NS):
                pltpu.sync_copy(
                    shared.at[pl.ds(t * 4 * HPAD + a * HPAD + s * HROWS_T,
                                    HROWS_T)], stg)

                def radd(i, _):
                    sl = pl.ds(i * 16, 16)
                    acc[sl] = acc[sl] + stg[sl]
                    return 0

                lax.fori_loop(0, HROWS_T // 16, radd, 0)

            off = (c * 4 + a) * HPAD + s * HROWS_T
            pltpu.sync_copy(acc, out.at[pl.ds(off, HROWS_T)])

    return k(s0, d0, s1, d1)


HALF = 16
GRP = 2
CH0 = 144
CH1 = 2 * NCH_T - CH0


def _sc_agg(xs0, xs1, s0, d0, s1, d1, zrows):
    mesh = _mesh()

    @functools.partial(
        pl.kernel,
        out_type=jax.ShapeDtypeStruct((4 * NPAD, D), jnp.float32),
        mesh=mesh,
        scratch_types=[
            pltpu.VMEM((HALF, CHUNK), jnp.int32),
            pltpu.VMEM((HALF, CHUNK), jnp.int32),
            pltpu.VMEM((CHUNK, D), jnp.float32),
            pltpu.VMEM((CHUNK, D), jnp.float32),
            pltpu.SemaphoreType.DMA,
            pltpu.SemaphoreType.DMA,
            pltpu.VMEM_SHARED((NPAD, D), jnp.float32),
        ],
    )
    def k(xs0r, xs1r, s0r, d0r, s1r, d1r, zr, out,
          sidx, didx, bufa, bufb, sema, semb, agg):
        bufs = (bufa, bufb)
        sems = (sema, semb)
        c = lax.axis_index("c")
        s = lax.axis_index("s")

        def run(tile_base, nch):
            for r, (xsr, sr, dr) in enumerate(
                    ((xs0r, s0r, d0r), (xs1r, s1r, d1r))):
                pltpu.sync_copy(zr, agg.at[pl.ds(s * ROWS_T, ROWS_T)])
                plsc.subcore_barrier()

                for h in range(nch // HALF):
                    base = tile_base + h * HALF
                    pltpu.sync_copy(sr.at[pl.ds(base, HALF)], sidx)
                    pltpu.sync_copy(dr.at[pl.ds(base, HALF)], didx)

                    def body(gg, _, xsr=xsr):
                        cps = []
                        for u in range(GRP):
                            j = gg * GRP + u
                            cps.append(pltpu.async_copy(
                                xsr.at[sidx.at[j]], bufs[u], sems[u]))
                        for u in range(GRP):
                            j = gg * GRP + u
                            cps[u].wait()
                            pltpu.sync_copy(bufs[u], agg.at[didx.at[j]],
                                            add=True)
                        return 0

                    lax.fori_loop(0, HALF // GRP, body, 0)
                plsc.subcore_barrier()

                off = (r * NC + c) * NPAD + s * ROWS_T
                pltpu.sync_copy(agg.at[pl.ds(s * ROWS_T, ROWS_T)],
                                out.at[pl.ds(off, ROWS_T)])
                if r == 0:
                    plsc.subcore_barrier()

        @pl.when(c == 0)
        def _():
            run(s * CH0, CH0)

        @pl.when(c == 1)
        def _():
            run(NS * CH0 + s * CH1, CH1)

    return k(xs0, xs1, s0, d0, s1, d1, zrows)


def _tc_scale(x, ht):
    def body(x_ref, ht_ref, xs0_ref, xs1_ref):
        xv = x_ref[...]
        od0 = ht_ref[:, 0:1] + ht_ref[:, 4:5]
        od1 = ht_ref[:, 2:3] + ht_ref[:, 6:7]
        isr0 = lax.rsqrt(jnp.maximum(od0, 1.0))[:N]
        isr1 = lax.rsqrt(jnp.maximum(od1, 1.0))[:N]
        xs0_ref[:N] = xv * isr0
        xs1_ref[:N] = xv * isr1
        zpad = jnp.zeros((NPAD - N, D), jnp.float32)
        xs0_ref[N:] = zpad
        xs1_ref[N:] = zpad

    return pl.pallas_call(
        body,
        out_shape=[
            jax.ShapeDtypeStruct((NPAD, D), jnp.float32),
            jax.ShapeDtypeStruct((NPAD, D), jnp.float32),
        ],
    )(x, ht)


def _tc_final(aggp, ht, W0, b0, W1, b1):
    def body(aggp_ref, ht_ref, w0_ref, b0_ref, w1_ref, b1_ref, out_ref):
        a0 = aggp_ref[0, :N, :] + aggp_ref[1, :N, :]
        a1 = aggp_ref[2, :N, :] + aggp_ref[3, :N, :]
        id0 = ht_ref[:, 1:2] + ht_ref[:, 5:6]
        id1 = ht_ref[:, 3:4] + ht_ref[:, 7:8]
        isr0 = lax.rsqrt(jnp.maximum(id0, 1.0))[:N]
        isr1 = lax.rsqrt(jnp.maximum(id1, 1.0))[:N]
        r0 = a0 * isr0
        r1 = a1 * isr1
        acc = jnp.dot(r0, w0_ref[...], preferred_element_type=jnp.float32)
        acc += jnp.dot(r1, w1_ref[...], preferred_element_type=jnp.float32)
        out_ref[...] = acc + (b0_ref[...] + b1_ref[...])[None, :]

    return pl.pallas_call(
        body,
        out_shape=jax.ShapeDtypeStruct((N, D), jnp.float32),
    )(aggp, ht, W0, b0, W1, b1)


def kernel(x, edge_index_rel0, edge_index_rel1, W0, b0, W1, b1):
    pad = jnp.full((EP - E,), TRASH, jnp.int32)

    def prep(v):
        return jnp.concatenate([v.astype(jnp.int32), pad]).reshape(
            NC * NS * NCH_T, CHUNK)

    s0 = prep(edge_index_rel0[0])
    d0 = prep(edge_index_rel0[1])
    s1 = prep(edge_index_rel1[0])
    d1 = prep(edge_index_rel1[1])

    zrows = jnp.zeros((ROWS_T, D), jnp.float32)

    hists = _sc_degrees(s0, d0, s1, d1)
    ht = hists.reshape(NC * 4, HPAD).T

    xs0, xs1 = _tc_scale(x, ht)

    aggp = _sc_agg(xs0, xs1, s0, d0, s1, d1, zrows)
    aggp = aggp.reshape(4, NPAD, D)

    return _tc_final(aggp, ht, W0, b0, W1, b1)


# speedup vs baseline: 1.5998x; 1.0033x over previous
import functools

import jax
import jax.numpy as jnp
from jax import lax
from jax.experimental import pallas as pl
from jax.experimental.pallas import tpu as pltpu
from jax.experimental.pallas import tpu_sc as plsc

N = 10000
D = 128
E = 320000

NC = 2
NS = 16
CHUNK = 128
NCH_T = 80
EP = NC * NS * NCH_T * CHUNK
TRASH = N

NPAD = 10112
ROWS_T = NPAD // NS
HPAD = 10240
HROWS_T = HPAD // NS


def _mesh():
    return plsc.VectorSubcoreMesh(core_axis_name="c", subcore_axis_name="s")


def _sc_degrees(s0, d0, s1, d1):
    mesh = _mesh()

    @functools.partial(
        pl.kernel,
        out_type=jax.ShapeDtypeStruct((NC * 4 * HPAD,), jnp.float32),
        mesh=mesh,
        compiler_params=pltpu.CompilerParams(needs_layout_passes=False),
        scratch_types=[
            pltpu.VMEM((NCH_T, CHUNK), jnp.int32),
            pltpu.VMEM((4 * HPAD,), jnp.float32),
            pltpu.VMEM((HROWS_T,), jnp.float32),
            pltpu.VMEM((HROWS_T,), jnp.float32),
            pltpu.VMEM_SHARED((NS * 4 * HPAD,), jnp.float32),
        ],
    )
    def k(s0r, d0r, s1r, d1r, out, idx_v, lhist, stg, acc, shared):
        c = lax.axis_index("c")
        s = lax.axis_index("s")
        g = c * NS + s

        zero16 = jnp.zeros((16,), jnp.float32)
        ones16 = jnp.ones((16,), jnp.float32)

        def zbody(i, _):
            lhist[pl.ds(i * 16, 16)] = zero16
            return 0

        lax.fori_loop(0, 4 * HPAD // 16, zbody, 0)

        for a, arr in enumerate((s0r, d0r, s1r, d1r)):
            pltpu.sync_copy(arr.at[pl.ds(g * NCH_T, NCH_T)], idx_v)

            def hbody(j, _, a=a):
                for kk in range(CHUNK // 16):
                    iv = idx_v[j, pl.ds(kk * 16, 16)] + (a * HPAD)
                    plsc.addupdate_scatter(lhist, [iv], ones16)
                return 0

            lax.fori_loop(0, NCH_T, hbody, 0)

        pltpu.sync_copy(lhist, shared.at[pl.ds(s * 4 * HPAD, 4 * HPAD)])
        plsc.subcore_barrier()

        for a in range(4):

            def zacc(i, _):
                acc[pl.ds(i * 16, 16)] = zero16
                return 0

            lax.fori_loop(0, HROWS_T // 16, zacc, 0)

            for t in range(NS):
                pltpu.sync_copy(
                    shared.at[pl.ds(t * 4 * HPAD + a * HPAD + s * HROWS_T,
                                    HROWS_T)], stg)

                def radd(i, _):
                    sl = pl.ds(i * 16, 16)
                    acc[sl] = acc[sl] + stg[sl]
                    return 0

                lax.fori_loop(0, HROWS_T // 16, radd, 0)

            off = (c * 4 + a) * HPAD + s * HROWS_T
            pltpu.sync_copy(acc, out.at[pl.ds(off, HROWS_T)])

    return k(s0, d0, s1, d1)


HALF = 8
GRP = 2
CH0 = 152
CH1 = 2 * NCH_T - CH0


def _sc_agg(xs0, xs1, s0, d0, s1, d1, zrows):
    mesh = _mesh()

    @functools.partial(
        pl.kernel,
        out_type=jax.ShapeDtypeStruct((4 * NPAD, D), jnp.float32),
        mesh=mesh,
        scratch_types=[
            pltpu.VMEM((HALF, CHUNK), jnp.int32),
            pltpu.VMEM((HALF, CHUNK), jnp.int32),
            pltpu.VMEM((CHUNK, D), jnp.float32),
            pltpu.VMEM((CHUNK, D), jnp.float32),
            pltpu.SemaphoreType.DMA,
            pltpu.SemaphoreType.DMA,
            pltpu.VMEM_SHARED((NPAD, D), jnp.float32),
        ],
    )
    def k(xs0r, xs1r, s0r, d0r, s1r, d1r, zr, out,
          sidx, didx, bufa, bufb, sema, semb, agg):
        bufs = (bufa, bufb)
        sems = (sema, semb)
        c = lax.axis_index("c")
        s = lax.axis_index("s")

        def run(tile_base, nch):
            for r, (xsr, sr, dr) in enumerate(
                    ((xs0r, s0r, d0r), (xs1r, s1r, d1r))):
                pltpu.sync_copy(zr, agg.at[pl.ds(s * ROWS_T, ROWS_T)])
                plsc.subcore_barrier()

                for h in range(nch // HALF):
                    base = tile_base + h * HALF
                    pltpu.sync_copy(sr.at[pl.ds(base, HALF)], sidx)
                    pltpu.sync_copy(dr.at[pl.ds(base, HALF)], didx)

                    def body(gg, _, xsr=xsr):
                        cps = []
                        for u in range(GRP):
                            j = gg * GRP + u
                            cps.append(pltpu.async_copy(
                                xsr.at[sidx.at[j]], bufs[u], sems[u]))
                        for u in range(GRP):
                            j = gg * GRP + u
                            cps[u].wait()
                            pltpu.sync_copy(bufs[u], agg.at[didx.at[j]],
                                            add=True)
                        return 0

                    lax.fori_loop(0, HALF // GRP, body, 0)
                plsc.subcore_barrier()

                off = (r * NC + c) * NPAD + s * ROWS_T
                pltpu.sync_copy(agg.at[pl.ds(s * ROWS_T, ROWS_T)],
                                out.at[pl.ds(off, ROWS_T)])
                if r == 0:
                    plsc.subcore_barrier()

        @pl.when(c == 0)
        def _():
            run(s * CH0, CH0)

        @pl.when(c == 1)
        def _():
            run(NS * CH0 + s * CH1, CH1)

    return k(xs0, xs1, s0, d0, s1, d1, zrows)


def _tc_scale(x, ht):
    def body(x_ref, ht_ref, xs0_ref, xs1_ref):
        xv = x_ref[...]
        od0 = ht_ref[:, 0:1] + ht_ref[:, 4:5]
        od1 = ht_ref[:, 2:3] + ht_ref[:, 6:7]
        isr0 = lax.rsqrt(jnp.maximum(od0, 1.0))[:N]
        isr1 = lax.rsqrt(jnp.maximum(od1, 1.0))[:N]
        xs0_ref[:N] = xv * isr0
        xs1_ref[:N] = xv * isr1
        zpad = jnp.zeros((NPAD - N, D), jnp.float32)
        xs0_ref[N:] = zpad
        xs1_ref[N:] = zpad

    return pl.pallas_call(
        body,
        out_shape=[
            jax.ShapeDtypeStruct((NPAD, D), jnp.float32),
            jax.ShapeDtypeStruct((NPAD, D), jnp.float32),
        ],
    )(x, ht)


def _tc_final(aggp, ht, W0, b0, W1, b1):
    def body(aggp_ref, ht_ref, w0_ref, b0_ref, w1_ref, b1_ref, out_ref):
        a0 = aggp_ref[0, :N, :] + aggp_ref[1, :N, :]
        a1 = aggp_ref[2, :N, :] + aggp_ref[3, :N, :]
        id0 = ht_ref[:, 1:2] + ht_ref[:, 5:6]
        id1 = ht_ref[:, 3:4] + ht_ref[:, 7:8]
        isr0 = lax.rsqrt(jnp.maximum(id0, 1.0))[:N]
        isr1 = lax.rsqrt(jnp.maximum(id1, 1.0))[:N]
        r0 = a0 * isr0
        r1 = a1 * isr1
        acc = jnp.dot(r0, w0_ref[...], preferred_element_type=jnp.float32)
        acc += jnp.dot(r1, w1_ref[...], preferred_element_type=jnp.float32)
        out_ref[...] = acc + (b0_ref[...] + b1_ref[...])[None, :]

    return pl.pallas_call(
        body,
        out_shape=jax.ShapeDtypeStruct((N, D), jnp.float32),
    )(aggp, ht, W0, b0, W1, b1)


def kernel(x, edge_index_rel0, edge_index_rel1, W0, b0, W1, b1):
    pad = jnp.full((EP - E,), TRASH, jnp.int32)

    def prep(v):
        return jnp.concatenate([v.astype(jnp.int32), pad]).reshape(
            NC * NS * NCH_T, CHUNK)

    s0 = prep(edge_index_rel0[0])
    d0 = prep(edge_index_rel0[1])
    s1 = prep(edge_index_rel1[0])
    d1 = prep(edge_index_rel1[1])

    zrows = jnp.zeros((ROWS_T, D), jnp.float32)

    hists = _sc_degrees(s0, d0, s1, d1)
    ht = hists.reshape(NC * 4, HPAD).T

    xs0, xs1 = _tc_scale(x, ht)

    aggp = _sc_agg(xs0, xs1, s0, d0, s1, d1, zrows)
    aggp = aggp.reshape(4, NPAD, D)

    return _tc_final(aggp, ht, W0, b0, W1, b1)


# speedup vs baseline: 1.6078x; 1.0050x over previous
import functools

import jax
import jax.numpy as jnp
from jax import lax
from jax.experimental import pallas as pl
from jax.experimental.pallas import tpu as pltpu
from jax.experimental.pallas import tpu_sc as plsc

N = 10000
D = 128
E = 320000

NC = 2
NS = 16
CHUNK = 128
NCH_T = 80
EP = NC * NS * NCH_T * CHUNK
TRASH = N

NPAD = 10112
ROWS_T = NPAD // NS
HPAD = 10240
HROWS_T = HPAD // NS


def _mesh():
    return plsc.VectorSubcoreMesh(core_axis_name="c", subcore_axis_name="s")


def _sc_degrees(s0, d0, s1, d1):
    mesh = _mesh()

    @functools.partial(
        pl.kernel,
        out_type=jax.ShapeDtypeStruct((NC * 4 * HPAD,), jnp.float32),
        mesh=mesh,
        compiler_params=pltpu.CompilerParams(needs_layout_passes=False),
        scratch_types=[
            pltpu.VMEM((DEG0, CHUNK), jnp.int32),
            pltpu.VMEM((4 * HPAD,), jnp.float32),
            pltpu.VMEM((HROWS_T,), jnp.float32),
            pltpu.VMEM((HROWS_T,), jnp.float32),
            pltpu.VMEM_SHARED((NS * 4 * HPAD,), jnp.float32),
        ],
    )
    def k(s0r, d0r, s1r, d1r, out, idx_v, lhist, stg, acc, shared):
        c = lax.axis_index("c")
        s = lax.axis_index("s")
        g = c * NS + s

        zero16 = jnp.zeros((16,), jnp.float32)
        ones16 = jnp.ones((16,), jnp.float32)

        def zbody(i, _):
            lhist[pl.ds(i * 16, 16)] = zero16
            return 0

        lax.fori_loop(0, 4 * HPAD // 16, zbody, 0)

        def hphase(tile_base, nch):
            for a, arr in enumerate((s0r, d0r, s1r, d1r)):
                pltpu.sync_copy(arr.at[pl.ds(tile_base, nch)],
                                idx_v.at[pl.ds(0, nch)])

                def hbody(j, _, a=a):
                    for kk in range(CHUNK // 16):
                        iv = idx_v[j, pl.ds(kk * 16, 16)] + (a * HPAD)
                        plsc.addupdate_scatter(lhist, [iv], ones16)
                    return 0

                lax.fori_loop(0, nch, hbody, 0)

        @pl.when(c == 0)
        def _():
            hphase(s * DEG0, DEG0)

        @pl.when(c == 1)
        def _():
            hphase(NS * DEG0 + s * DEG1, DEG1)

        pltpu.sync_copy(lhist, shared.at[pl.ds(s * 4 * HPAD, 4 * HPAD)])
        plsc.subcore_barrier()

        for a in range(4):

            def zacc(i, _):
                acc[pl.ds(i * 16, 16)] = zero16
                return 0

            lax.fori_loop(0, HROWS_T // 16, zacc, 0)

            for t in range(NS):
                pltpu.sync_copy(
                    shared.at[pl.ds(t * 4 * HPAD + a * HPAD + s * HROWS_T,
                                    HROWS_T)], stg)

                def radd(i, _):
                    sl = pl.ds(i * 16, 16)
                    acc[sl] = acc[sl] + stg[sl]
                    return 0

                lax.fori_loop(0, HROWS_T // 16, radd, 0)

            off = (c * 4 + a) * HPAD + s * HROWS_T
            pltpu.sync_copy(acc, out.at[pl.ds(off, HROWS_T)])

    return k(s0, d0, s1, d1)


HALF = 8
GRP = 2
CH0 = 152
CH1 = 2 * NCH_T - CH0
DEG0 = 96
DEG1 = 2 * NCH_T - DEG0


def _sc_agg(xs0, xs1, s0, d0, s1, d1, zrows):
    mesh = _mesh()

    @functools.partial(
        pl.kernel,
        out_type=jax.ShapeDtypeStruct((4 * NPAD, D), jnp.float32),
        mesh=mesh,
        scratch_types=[
            pltpu.VMEM((HALF, CHUNK), jnp.int32),
            pltpu.VMEM((HALF, CHUNK), jnp.int32),
            pltpu.VMEM((CHUNK, D), jnp.float32),
            pltpu.VMEM((CHUNK, D), jnp.float32),
            pltpu.SemaphoreType.DMA,
            pltpu.SemaphoreType.DMA,
            pltpu.VMEM_SHARED((NPAD, D), jnp.float32),
        ],
    )
    def k(xs0r, xs1r, s0r, d0r, s1r, d1r, zr, out,
          sidx, didx, bufa, bufb, sema, semb, agg):
        bufs = (bufa, bufb)
        sems = (sema, semb)
        c = lax.axis_index("c")
        s = lax.axis_index("s")

        def run(tile_base, nch):
            for r, (xsr, sr, dr) in enumerate(
                    ((xs0r, s0r, d0r), (xs1r, s1r, d1r))):
                pltpu.sync_copy(zr, agg.at[pl.ds(s * ROWS_T, ROWS_T)])
                plsc.subcore_barrier()

                for h in range(nch // HALF):
                    base = tile_base + h * HALF
                    pltpu.sync_copy(sr.at[pl.ds(base, HALF)], sidx)
                    pltpu.sync_copy(dr.at[pl.ds(base, HALF)], didx)

                    def body(gg, _, xsr=xsr):
                        cps = []
                        for u in range(GRP):
                            j = gg * GRP + u
                            cps.append(pltpu.async_copy(
                                xsr.at[sidx.at[j]], bufs[u], sems[u]))
                        for u in range(GRP):
                            j = gg * GRP + u
                            cps[u].wait()
                            pltpu.sync_copy(bufs[u], agg.at[didx.at[j]],
                                            add=True)
                        return 0

                    lax.fori_loop(0, HALF // GRP, body, 0)
                plsc.subcore_barrier()

                off = (r * NC + c) * NPAD + s * ROWS_T
                pltpu.sync_copy(agg.at[pl.ds(s * ROWS_T, ROWS_T)],
                                out.at[pl.ds(off, ROWS_T)])
                if r == 0:
                    plsc.subcore_barrier()

        @pl.when(c == 0)
        def _():
            run(s * CH0, CH0)

        @pl.when(c == 1)
        def _():
            run(NS * CH0 + s * CH1, CH1)

    return k(xs0, xs1, s0, d0, s1, d1, zrows)


def _tc_scale(x, ht):
    def body(x_ref, ht_ref, xs0_ref, xs1_ref):
        xv = x_ref[...]
        od0 = ht_ref[:, 0:1] + ht_ref[:, 4:5]
        od1 = ht_ref[:, 2:3] + ht_ref[:, 6:7]
        isr0 = lax.rsqrt(jnp.maximum(od0, 1.0))[:N]
        isr1 = lax.rsqrt(jnp.maximum(od1, 1.0))[:N]
        xs0_ref[:N] = xv * isr0
        xs1_ref[:N] = xv * isr1
        zpad = jnp.zeros((NPAD - N, D), jnp.float32)
        xs0_ref[N:] = zpad
        xs1_ref[N:] = zpad

    return pl.pallas_call(
        body,
        out_shape=[
            jax.ShapeDtypeStruct((NPAD, D), jnp.float32),
            jax.ShapeDtypeStruct((NPAD, D), jnp.float32),
        ],
    )(x, ht)


def _tc_final(aggp, ht, W0, b0, W1, b1):
    def body(aggp_ref, ht_ref, w0_ref, b0_ref, w1_ref, b1_ref, out_ref):
        a0 = aggp_ref[0, :N, :] + aggp_ref[1, :N, :]
        a1 = aggp_ref[2, :N, :] + aggp_ref[3, :N, :]
        id0 = ht_ref[:, 1:2] + ht_ref[:, 5:6]
        id1 = ht_ref[:, 3:4] + ht_ref[:, 7:8]
        isr0 = lax.rsqrt(jnp.maximum(id0, 1.0))[:N]
        isr1 = lax.rsqrt(jnp.maximum(id1, 1.0))[:N]
        r0 = a0 * isr0
        r1 = a1 * isr1
        acc = jnp.dot(r0, w0_ref[...], preferred_element_type=jnp.float32)
        acc += jnp.dot(r1, w1_ref[...], preferred_element_type=jnp.float32)
        out_ref[...] = acc + (b0_ref[...] + b1_ref[...])[None, :]

    return pl.pallas_call(
        body,
        out_shape=jax.ShapeDtypeStruct((N, D), jnp.float32),
    )(aggp, ht, W0, b0, W1, b1)


def kernel(x, edge_index_rel0, edge_index_rel1, W0, b0, W1, b1):
    pad = jnp.full((EP - E,), TRASH, jnp.int32)

    def prep(v):
        return jnp.concatenate([v.astype(jnp.int32), pad]).reshape(
            NC * NS * NCH_T, CHUNK)

    s0 = prep(edge_index_rel0[0])
    d0 = prep(edge_index_rel0[1])
    s1 = prep(edge_index_rel1[0])
    d1 = prep(edge_index_rel1[1])

    zrows = jnp.zeros((ROWS_T, D), jnp.float32)

    hists = _sc_degrees(s0, d0, s1, d1)
    ht = hists.reshape(NC * 4, HPAD).T

    xs0, xs1 = _tc_scale(x, ht)

    aggp = _sc_agg(xs0, xs1, s0, d0, s1, d1, zrows)
    aggp = aggp.reshape(4, NPAD, D)

    return _tc_final(aggp, ht, W0, b0, W1, b1)


# speedup vs baseline: 1.6087x; 1.0005x over previous
import functools

import jax
import jax.numpy as jnp
from jax import lax
from jax.experimental import pallas as pl
from jax.experimental.pallas import tpu as pltpu
from jax.experimental.pallas import tpu_sc as plsc

N = 10000
D = 128
E = 320000

NC = 2
NS = 16
CHUNK = 128
NCH_T = 80
EP = NC * NS * NCH_T * CHUNK
TRASH = N

NPAD = 10112
ROWS_T = NPAD // NS
HPAD = 10240
HROWS_T = HPAD // NS


def _mesh():
    return plsc.VectorSubcoreMesh(core_axis_name="c", subcore_axis_name="s")


def _sc_degrees(s0, d0, s1, d1):
    mesh = _mesh()

    @functools.partial(
        pl.kernel,
        out_type=jax.ShapeDtypeStruct((NC * 4 * HPAD,), jnp.float32),
        mesh=mesh,
        compiler_params=pltpu.CompilerParams(needs_layout_passes=False),
        scratch_types=[
            pltpu.VMEM((DEG0, CHUNK), jnp.int32),
            pltpu.VMEM((4 * HPAD,), jnp.float32),
            pltpu.VMEM((HROWS_T,), jnp.float32),
            pltpu.VMEM((HROWS_T,), jnp.float32),
            pltpu.VMEM_SHARED((NS * 4 * HPAD,), jnp.float32),
        ],
    )
    def k(s0r, d0r, s1r, d1r, out, idx_v, lhist, stg, acc, shared):
        c = lax.axis_index("c")
        s = lax.axis_index("s")

        zero16 = jnp.zeros((16,), jnp.float32)
        ones16 = jnp.ones((16,), jnp.float32)

        def zbody(i, _):
            lhist[pl.ds(i * 16, 16)] = zero16
            return 0

        lax.fori_loop(0, 4 * HPAD // 16, zbody, 0)

        def hphase(tile_base, nch):
            for a, arr in enumerate((s0r, d0r, s1r, d1r)):
                pltpu.sync_copy(arr.at[pl.ds(tile_base, nch)],
                                idx_v.at[pl.ds(0, nch)])

                def hbody(j, _, a=a):
                    for kk in range(CHUNK // 16):
                        iv = idx_v[j, pl.ds(kk * 16, 16)] + (a * HPAD)
                        plsc.addupdate_scatter(lhist, [iv], ones16)
                    return 0

                lax.fori_loop(0, nch, hbody, 0)

        @pl.when(c == 0)
        def _():
            hphase(s * DEG0, DEG0)

        @pl.when(c == 1)
        def _():
            hphase(NS * DEG0 + s * DEG1, DEG1)

        pltpu.sync_copy(lhist, shared.at[pl.ds(s * 4 * HPAD, 4 * HPAD)])
        plsc.subcore_barrier()

        for a in range(4):

            def zacc(i, _):
                acc[pl.ds(i * 16, 16)] = zero16
                return 0

            lax.fori_loop(0, HROWS_T // 16, zacc, 0)

            for t in range(NS):
                pltpu.sync_copy(
                    shared.at[pl.ds(t * 4 * HPAD + a * HPAD + s * HROWS_T,
                                    HROWS_T)], stg)

                def radd(i, _):
                    sl = pl.ds(i * 16, 16)
                    acc[sl] = acc[sl] + stg[sl]
                    return 0

                lax.fori_loop(0, HROWS_T // 16, radd, 0)

            off = (c * 4 + a) * HPAD + s * HROWS_T
            pltpu.sync_copy(acc, out.at[pl.ds(off, HROWS_T)])

    return k(s0, d0, s1, d1)


HALF = 8
GRP = 2
CH0 = 152
CH1 = 2 * NCH_T - CH0
DEG0 = 96
DEG1 = 2 * NCH_T - DEG0


def _sc_agg(xs0, xs1, s0, d0, s1, d1, zrows):
    mesh = _mesh()

    @functools.partial(
        pl.kernel,
        out_type=jax.ShapeDtypeStruct((4 * NPAD, D), jnp.float32),
        mesh=mesh,
        scratch_types=[
            pltpu.VMEM((HALF, CHUNK), jnp.int32),
            pltpu.VMEM((HALF, CHUNK), jnp.int32),
            pltpu.VMEM((CHUNK, D), jnp.float32),
            pltpu.VMEM((CHUNK, D), jnp.float32),
            pltpu.SemaphoreType.DMA,
            pltpu.SemaphoreType.DMA,
            pltpu.VMEM_SHARED((NPAD, D), jnp.float32),
        ],
    )
    def k(xs0r, xs1r, s0r, d0r, s1r, d1r, zr, out,
          sidx, didx, bufa, bufb, sema, semb, agg):
        bufs = (bufa, bufb)
        sems = (sema, semb)
        c = lax.axis_index("c")
        s = lax.axis_index("s")

        def run(tile_base, nch):
            for r, (xsr, sr, dr) in enumerate(
                    ((xs0r, s0r, d0r), (xs1r, s1r, d1r))):
                pltpu.sync_copy(zr, agg.at[pl.ds(s * ROWS_T, ROWS_T)])
                plsc.subcore_barrier()

                for h in range(nch // HALF):
                    base = tile_base + h * HALF
                    pltpu.sync_copy(sr.at[pl.ds(base, HALF)], sidx)
                    pltpu.sync_copy(dr.at[pl.ds(base, HALF)], didx)

                    def body(gg, _, xsr=xsr):
                        cps = []
                        for u in range(GRP):
                            j = gg * GRP + u
                            cps.append(pltpu.async_copy(
                                xsr.at[sidx.at[j]], bufs[u], sems[u]))
                        for u in range(GRP):
                            j = gg * GRP + u
                            cps[u].wait()
                            pltpu.sync_copy(bufs[u], agg.at[didx.at[j]],
                                            add=True)
                        return 0

                    lax.fori_loop(0, HALF // GRP, body, 0)
                plsc.subcore_barrier()

                off = (r * NC + c) * NPAD + s * ROWS_T
                pltpu.sync_copy(agg.at[pl.ds(s * ROWS_T, ROWS_T)],
                                out.at[pl.ds(off, ROWS_T)])
                if r == 0:
                    plsc.subcore_barrier()

        @pl.when(c == 0)
        def _():
            run(s * CH0, CH0)

        @pl.when(c == 1)
        def _():
            run(NS * CH0 + s * CH1, CH1)

    return k(xs0, xs1, s0, d0, s1, d1, zrows)


def _tc_scale(x, ht):
    def body(x_ref, ht_ref, xs0_ref, xs1_ref):
        xv = x_ref[...]
        od0 = ht_ref[:, 0:1] + ht_ref[:, 4:5]
        od1 = ht_ref[:, 2:3] + ht_ref[:, 6:7]
        isr0 = lax.rsqrt(jnp.maximum(od0, 1.0))[:N]
        isr1 = lax.rsqrt(jnp.maximum(od1, 1.0))[:N]
        xs0_ref[:N] = xv * isr0
        xs1_ref[:N] = xv * isr1
        zpad = jnp.zeros((NPAD - N, D), jnp.float32)
        xs0_ref[N:] = zpad
        xs1_ref[N:] = zpad

    return pl.pallas_call(
        body,
        out_shape=[
            jax.ShapeDtypeStruct((NPAD, D), jnp.float32),
            jax.ShapeDtypeStruct((NPAD, D), jnp.float32),
        ],
    )(x, ht)


def _tc_final(aggp, ht, W0, b0, W1, b1):
    def body(aggp_ref, ht_ref, w0_ref, b0_ref, w1_ref, b1_ref, out_ref):
        a0 = aggp_ref[0, :N, :] + aggp_ref[1, :N, :]
        a1 = aggp_ref[2, :N, :] + aggp_ref[3, :N, :]
        id0 = ht_ref[:, 1:2] + ht_ref[:, 5:6]
        id1 = ht_ref[:, 3:4] + ht_ref[:, 7:8]
        isr0 = lax.rsqrt(jnp.maximum(id0, 1.0))[:N]
        isr1 = lax.rsqrt(jnp.maximum(id1, 1.0))[:N]
        r0 = a0 * isr0
        r1 = a1 * isr1
        acc = jnp.dot(r0, w0_ref[...], preferred_element_type=jnp.float32)
        acc += jnp.dot(r1, w1_ref[...], preferred_element_type=jnp.float32)
        out_ref[...] = acc + (b0_ref[...] + b1_ref[...])[None, :]

    return pl.pallas_call(
        body,
        out_shape=jax.ShapeDtypeStruct((N, D), jnp.float32),
    )(aggp, ht, W0, b0, W1, b1)


def kernel(x, edge_index_rel0, edge_index_rel1, W0, b0, W1, b1):
    pad = jnp.full((EP - E,), TRASH, jnp.int32)

    def prep(v):
        return jnp.concatenate([v.astype(jnp.int32), pad]).reshape(
            NC * NS * NCH_T, CHUNK)

    s0 = prep(edge_index_rel0[0])
    d0 = prep(edge_index_rel0[1])
    s1 = prep(edge_index_rel1[0])
    d1 = prep(edge_index_rel1[1])

    zrows = jnp.zeros((ROWS_T, D), jnp.float32)

    hists = _sc_degrees(s0, d0, s1, d1)
    ht = hists.reshape(NC * 4, HPAD).T

    xs0, xs1 = _tc_scale(x, ht)

    aggp = _sc_agg(xs0, xs1, s0, d0, s1, d1, zrows)
    aggp = aggp.reshape(4, NPAD, D)

    return _tc_final(aggp, ht, W0, b0, W1, b1)
